# baseline (speedup 1.0000x reference)
import functools

import jax
import jax.numpy as jnp
from jax import lax
from jax.experimental import pallas as pl
from jax.experimental.pallas import tpu as pltpu
from jax.experimental.pallas import tpu_sc as plsc

B, S, D, E, F = 1, 2048, 1024, 8, 2048
C = 512
SECOND_THRESHOLD = 0.2
LOSS_COEF = 0.01
NSLOT = E * C
SENT = NSLOT
NROWS = NSLOT + 8

_NC, _NS = 2, 16
_NW = _NC * _NS
_TPW = S // _NW


def _routing_body(x_ref, wg_ref, slot1_ref, slot2_ref, g1_ref, g2_ref,
                  loss_ref):
    x = x_ref[...]
    wgt = wg_ref[...]
    logits = lax.dot_general(x, wgt, (((1,), (1,)), ((), ())),
                             preferred_element_type=jnp.float32)
    m = jnp.max(logits, axis=-1, keepdims=True)
    ex = jnp.exp(logits - m)
    raw = ex / jnp.sum(ex, axis=-1, keepdims=True)

    e_iota = lax.broadcasted_iota(jnp.int32, (S, E), 1)
    gate1 = jnp.max(raw, axis=-1, keepdims=True)
    idx1 = jnp.min(jnp.where(raw == gate1, e_iota, E), axis=-1,
                   keepdims=True)
    mask1 = (e_iota == idx1).astype(jnp.float32)
    raw_wo1 = raw * (1.0 - mask1)
    gate2 = jnp.max(raw_wo1, axis=-1, keepdims=True)
    idx2 = jnp.min(jnp.where(raw_wo1 == gate2, e_iota, E), axis=-1,
                   keepdims=True)
    mask2 = (e_iota == idx2).astype(jnp.float32)

    denom = gate1 + gate2 + 1e-9
    g1 = gate1 / denom
    g2 = gate2 / denom
    mask2 = mask2 * (g2 > SECOND_THRESHOLD).astype(jnp.float32)

    dp = jnp.mean(raw, axis=0, keepdims=True)
    d1 = jnp.mean(mask1, axis=0, keepdims=True)
    loss_ref[...] = (dp * d1 * (float(E * E) / E) * LOSS_COEF).sum(
        axis=-1, keepdims=True)

    def excl_cumsum(m):
        inc = m
        k = 1
        while k < S:
            inc = inc + jnp.concatenate(
                [jnp.zeros((k, E), jnp.float32), inc[:-k]], axis=0)
            k *= 2
        return inc - m

    pos1 = excl_cumsum(mask1)
    mask1c = mask1 * (pos1 < C).astype(jnp.float32)
    m1cnt = jnp.sum(mask1c, axis=0, keepdims=True)
    pos2 = excl_cumsum(mask2) + m1cnt
    mask2c = mask2 * (pos2 < C).astype(jnp.float32)

    in1 = jnp.sum(mask1c, axis=-1, keepdims=True)
    in2 = jnp.sum(mask2c, axis=-1, keepdims=True)
    p1 = jnp.sum(pos1 * mask1c, axis=-1, keepdims=True).astype(jnp.int32)
    p2 = jnp.sum(pos2 * mask2c, axis=-1, keepdims=True).astype(jnp.int32)
    slot1_ref[...] = jnp.where(in1 > 0.0, idx1 * C + p1, SENT).reshape(S)
    slot2_ref[...] = jnp.where(in2 > 0.0, idx2 * C + p2, SENT).reshape(S)
    g1_ref[...] = jnp.broadcast_to(g1 * in1, (S, 16))
    g2_ref[...] = jnp.broadcast_to(g2 * in2, (S, 16))


def _routing(x, wgt):
    return pl.pallas_call(
        _routing_body,
        out_shape=(
            jax.ShapeDtypeStruct((S,), jnp.int32),
            jax.ShapeDtypeStruct((S,), jnp.int32),
            jax.ShapeDtypeStruct((S, 16), jnp.float32),
            jax.ShapeDtypeStruct((S, 16), jnp.float32),
            jax.ShapeDtypeStruct((1, 1), jnp.float32),
        ),
    )(x, wgt)


@functools.cache
def _sc_kernels():
    mesh = plsc.VectorSubcoreMesh(core_axis_name="c", subcore_axis_name="s")
    scratch = [
        pltpu.VMEM((_TPW,), jnp.int32),
        pltpu.VMEM((_TPW, D), jnp.float32),
        pltpu.SemaphoreType.DMA,
    ]

    @functools.partial(
        pl.kernel,
        mesh=mesh,
        out_type=jax.ShapeDtypeStruct((NROWS, D), jnp.float32),
        scratch_types=scratch,
    )
    def dispatch(x_hbm, slot1_hbm, slot2_hbm, expin_hbm, idx_v, rows_v, sem):
        wid = lax.axis_index("s") * _NC + lax.axis_index("c")
        base = wid * _TPW
        pltpu.sync_copy(x_hbm.at[pl.ds(base, _TPW)], rows_v)
        pltpu.sync_copy(slot1_hbm.at[pl.ds(base, _TPW)], idx_v)
        pltpu.async_copy(rows_v, expin_hbm.at[idx_v], sem).wait()
        pltpu.sync_copy(slot2_hbm.at[pl.ds(base, _TPW)], idx_v)
        pltpu.async_copy(rows_v, expin_hbm.at[idx_v], sem).wait()

    QT = 16
    NQ = _TPW // QT

    @functools.partial(
        pl.kernel,
        mesh=mesh,
        out_type=jax.ShapeDtypeStruct((S, D), jnp.float32),
        scratch_types=[
            pltpu.VMEM((_TPW,), jnp.int32),
            pltpu.VMEM((_TPW,), jnp.int32),
            pltpu.VMEM((_TPW, 16), jnp.float32),
            pltpu.VMEM((_TPW, 16), jnp.float32),
            pltpu.VMEM((2, QT, D), jnp.float32),
            pltpu.VMEM((2, QT, D), jnp.float32),
            pltpu.VMEM((2, QT, D), jnp.float32),
            pltpu.SemaphoreType.DMA,
            pltpu.SemaphoreType.DMA,
            pltpu.SemaphoreType.DMA,
            pltpu.SemaphoreType.DMA,
            pltpu.SemaphoreType.DMA,
            pltpu.SemaphoreType.DMA,
        ],
    )
    def combine(expout_hbm, slot1_hbm, slot2_hbm, g1_hbm, g2_hbm, out_hbm,
                idx1_v, idx2_v, g1_v, g2_v, rows1_v, rows2_v, outq_v,
                s1a, s1b, s2a, s2b, soa, sob):
        wid = lax.axis_index("s") * _NC + lax.axis_index("c")
        base = wid * _TPW
        pltpu.sync_copy(slot1_hbm.at[pl.ds(base, _TPW)], idx1_v)
        pltpu.sync_copy(slot2_hbm.at[pl.ds(base, _TPW)], idx2_v)
        pltpu.sync_copy(g1_hbm.at[pl.ds(base, _TPW)], g1_v)
        pltpu.sync_copy(g2_hbm.at[pl.ds(base, _TPW)], g2_v)
        sem1 = (s1a, s1b)
        sem2 = (s2a, s2b)
        semo = (soa, sob)

        def issue(q):
            b = q % 2
            a1 = pltpu.async_copy(
                expout_hbm.at[idx1_v.at[pl.ds(q * QT, QT)]],
                rows1_v.at[b], sem1[b])
            a2 = pltpu.async_copy(
                expout_hbm.at[idx2_v.at[pl.ds(q * QT, QT)]],
                rows2_v.at[b], sem2[b])
            return a1, a2

        pend = issue(0)
        owrites = [None, None]
        for q in range(NQ):
            nxt = issue(q + 1) if q + 1 < NQ else None
            pend[0].wait()
            pend[1].wait()
            b = q % 2
            if owrites[b] is not None:
                owrites[b].wait()
                owrites[b] = None

            def body(tl, _):
                t = q * QT + tl
                g1s = g1_v[t, :]
                g2s = g2_v[t, :]
                zero = jnp.zeros((16,), jnp.float32)

                def inner(j, _2):
                    for u in range(8):
                        off = (j * 8 + u) * 16
                        r1 = rows1_v[b, tl, pl.ds(off, 16)]
                        r2 = rows2_v[b, tl, pl.ds(off, 16)]
                        v = (jnp.where(g1s > 0.0, r1 * g1s, zero)
                             + jnp.where(g2s > 0.0, r2 * g2s, zero))
                        outq_v[b, tl, pl.ds(off, 16)] = v
                    return _2

                return lax.fori_loop(0, D // 128, inner, _)

            lax.fori_loop(0, QT, body, 0)
            owrites[b] = pltpu.async_copy(
                outq_v.at[b], out_hbm.at[pl.ds(base + q * QT, QT)], semo[b])
            pend = nxt
        for w in owrites:
            if w is not None:
                w.wait()

    return dispatch, combine


def _ffn_body(xin_ref, w1_ref, b1_ref, w2_ref, b2_ref, out_ref):
    e = pl.program_id(0)
    xin = xin_ref[...]
    h = jnp.dot(xin, w1_ref[0], preferred_element_type=jnp.float32)
    h = jnp.maximum(h + b1_ref[pl.ds(e, 1), :], 0.0)
    out = jnp.dot(h, w2_ref[0], preferred_element_type=jnp.float32)
    out_ref[...] = out + b2_ref[pl.ds(e, 1), :]


def _ffn(exp_in, w1, b1, w2, b2):
    return pl.pallas_call(
        _ffn_body,
        grid=(E,),
        in_specs=[
            pl.BlockSpec((C, D), lambda e: (e, 0)),
            pl.BlockSpec((1, D, F), lambda e: (e, 0, 0)),
            pl.BlockSpec((E, F), lambda e: (0, 0)),
            pl.BlockSpec((1, F, D), lambda e: (e, 0, 0)),
            pl.BlockSpec((E, D), lambda e: (0, 0)),
        ],
        out_specs=pl.BlockSpec((C, D), lambda e: (e, 0)),
        out_shape=jax.ShapeDtypeStruct((NROWS, D), jnp.float32),
        compiler_params=pltpu.CompilerParams(
            vmem_limit_bytes=100 * 1024 * 1024),
    )(exp_in, w1, b1, w2, b2)


def kernel(hidden_states, Wg, W1, b1, W2, b2):
    x = hidden_states.reshape(S, D)
    s1, s2, g1, g2, loss = _routing(x, Wg.T)
    dispatch, combine = _sc_kernels()
    exp_in = dispatch(x, s1, s2)
    exp_out = _ffn(exp_in, W1, b1, W2, b2)
    out = combine(exp_out, s1, s2, g1, g2)
    return out.reshape(B, S, D), loss[0, 0]


# speedup vs baseline: 1.2323x; 1.2323x over previous
import functools

import jax
import jax.numpy as jnp
from jax import lax
from jax.experimental import pallas as pl
from jax.experimental.pallas import tpu as pltpu
from jax.experimental.pallas import tpu_sc as plsc

B, S, D, E, F = 1, 2048, 1024, 8, 2048
C = 512
SECOND_THRESHOLD = 0.2
LOSS_COEF = 0.01
NSLOT = E * C
SENT = NSLOT
NROWS = NSLOT + 8

_NC, _NS = 2, 16
_NW = _NC * _NS
_TPW = S // _NW


def _routing_body(x_ref, wg_ref, slot1_ref, slot2_ref, g1_ref, g2_ref,
                  loss_ref):
    x = x_ref[...]
    wgt = wg_ref[...]
    logits = lax.dot_general(wgt, x, (((1,), (1,)), ((), ())),
                             preferred_element_type=jnp.float32)
    m = jnp.max(logits, axis=0, keepdims=True)
    ex = jnp.exp(logits - m)
    raw = ex / jnp.sum(ex, axis=0, keepdims=True)

    e_iota = lax.broadcasted_iota(jnp.int32, (E, S), 0)
    gate1 = jnp.max(raw, axis=0, keepdims=True)
    idx1 = jnp.min(jnp.where(raw == gate1, e_iota, E), axis=0,
                   keepdims=True)
    mask1 = (e_iota == idx1).astype(jnp.float32)
    raw_wo1 = raw * (1.0 - mask1)
    gate2 = jnp.max(raw_wo1, axis=0, keepdims=True)
    idx2 = jnp.min(jnp.where(raw_wo1 == gate2, e_iota, E), axis=0,
                   keepdims=True)
    mask2 = (e_iota == idx2).astype(jnp.float32)

    denom = gate1 + gate2 + 1e-9
    g1 = gate1 / denom
    g2 = gate2 / denom
    mask2 = mask2 * (g2 > SECOND_THRESHOLD).astype(jnp.float32)

    dp = jnp.mean(raw, axis=1, keepdims=True)
    d1 = jnp.mean(mask1, axis=1, keepdims=True)
    loss_ref[...] = jnp.sum(dp * d1 * (float(E * E) / E) * LOSS_COEF,
                            axis=0, keepdims=True)

    def excl_cumsum(mk):
        inc = mk
        k = 1
        while k < S:
            inc = inc + jnp.concatenate(
                [jnp.zeros((E, k), jnp.float32), inc[:, :-k]], axis=1)
            k *= 2
        return inc - mk

    pos1 = excl_cumsum(mask1)
    mask1c = mask1 * (pos1 < C).astype(jnp.float32)
    m1cnt = jnp.sum(mask1c, axis=1, keepdims=True)
    pos2 = excl_cumsum(mask2) + m1cnt
    mask2c = mask2 * (pos2 < C).astype(jnp.float32)

    in1 = jnp.sum(mask1c, axis=0, keepdims=True)
    in2 = jnp.sum(mask2c, axis=0, keepdims=True)
    p1 = jnp.sum(pos1 * mask1c, axis=0, keepdims=True).astype(jnp.int32)
    p2 = jnp.sum(pos2 * mask2c, axis=0, keepdims=True).astype(jnp.int32)
    slot1_ref[...] = jnp.where(in1 > 0.0, idx1 * C + p1, SENT).reshape(S)
    slot2_ref[...] = jnp.where(in2 > 0.0, idx2 * C + p2, SENT).reshape(S)
    g1_ref[...] = jnp.broadcast_to((g1 * in1).reshape(S, 1), (S, 16))
    g2_ref[...] = jnp.broadcast_to((g2 * in2).reshape(S, 1), (S, 16))


def _routing(x, wgt):
    return pl.pallas_call(
        _routing_body,
        out_shape=(
            jax.ShapeDtypeStruct((S,), jnp.int32),
            jax.ShapeDtypeStruct((S,), jnp.int32),
            jax.ShapeDtypeStruct((S, 16), jnp.float32),
            jax.ShapeDtypeStruct((S, 16), jnp.float32),
            jax.ShapeDtypeStruct((1, 1), jnp.float32),
        ),
    )(x, wgt)


@functools.cache
def _sc_kernels():
    mesh = plsc.VectorSubcoreMesh(core_axis_name="c", subcore_axis_name="s")
    scratch = [
        pltpu.VMEM((_TPW,), jnp.int32),
        pltpu.VMEM((_TPW, D), jnp.float32),
        pltpu.SemaphoreType.DMA,
    ]

    @functools.partial(
        pl.kernel,
        mesh=mesh,
        out_type=jax.ShapeDtypeStruct((NROWS, D), jnp.float32),
        scratch_types=scratch,
    )
    def dispatch(x_hbm, slot1_hbm, slot2_hbm, expin_hbm, idx_v, rows_v, sem):
        wid = lax.axis_index("s") * _NC + lax.axis_index("c")
        base = wid * _TPW
        pltpu.sync_copy(x_hbm.at[pl.ds(base, _TPW)], rows_v)
        pltpu.sync_copy(slot1_hbm.at[pl.ds(base, _TPW)], idx_v)
        pltpu.async_copy(rows_v, expin_hbm.at[idx_v], sem).wait()
        pltpu.sync_copy(slot2_hbm.at[pl.ds(base, _TPW)], idx_v)
        pltpu.async_copy(rows_v, expin_hbm.at[idx_v], sem).wait()

    QT = 16
    NQ = _TPW // QT

    @functools.partial(
        pl.kernel,
        mesh=mesh,
        out_type=jax.ShapeDtypeStruct((S, D), jnp.float32),
        scratch_types=[
            pltpu.VMEM((_TPW,), jnp.int32),
            pltpu.VMEM((_TPW,), jnp.int32),
            pltpu.VMEM((_TPW, 16), jnp.float32),
            pltpu.VMEM((_TPW, 16), jnp.float32),
            pltpu.VMEM((2, QT, D), jnp.float32),
            pltpu.VMEM((2, QT, D), jnp.float32),
            pltpu.VMEM((2, QT, D), jnp.float32),
            pltpu.SemaphoreType.DMA,
            pltpu.SemaphoreType.DMA,
            pltpu.SemaphoreType.DMA,
            pltpu.SemaphoreType.DMA,
            pltpu.SemaphoreType.DMA,
            pltpu.SemaphoreType.DMA,
        ],
    )
    def combine(expout_hbm, slot1_hbm, slot2_hbm, g1_hbm, g2_hbm, out_hbm,
                idx1_v, idx2_v, g1_v, g2_v, rows1_v, rows2_v, outq_v,
                s1a, s1b, s2a, s2b, soa, sob):
        wid = lax.axis_index("s") * _NC + lax.axis_index("c")
        base = wid * _TPW
        pltpu.sync_copy(slot1_hbm.at[pl.ds(base, _TPW)], idx1_v)
        pltpu.sync_copy(slot2_hbm.at[pl.ds(base, _TPW)], idx2_v)
        pltpu.sync_copy(g1_hbm.at[pl.ds(base, _TPW)], g1_v)
        pltpu.sync_copy(g2_hbm.at[pl.ds(base, _TPW)], g2_v)
        sem1 = (s1a, s1b)
        sem2 = (s2a, s2b)
        semo = (soa, sob)

        def issue(q):
            b = q % 2
            a1 = pltpu.async_copy(
                expout_hbm.at[idx1_v.at[pl.ds(q * QT, QT)]],
                rows1_v.at[b], sem1[b])
            a2 = pltpu.async_copy(
                expout_hbm.at[idx2_v.at[pl.ds(q * QT, QT)]],
                rows2_v.at[b], sem2[b])
            return a1, a2

        pend = issue(0)
        owrites = [None, None]
        for q in range(NQ):
            nxt = issue(q + 1) if q + 1 < NQ else None
            pend[0].wait()
            pend[1].wait()
            b = q % 2
            if owrites[b] is not None:
                owrites[b].wait()
                owrites[b] = None

            def body(tl, _):
                t = q * QT + tl
                g1s = g1_v[t, :]
                g2s = g2_v[t, :]
                zero = jnp.zeros((16,), jnp.float32)

                def inner(j, _2):
                    for u in range(4):
                        off = (j * 4 + u) * 16
                        r1 = rows1_v[b, tl, pl.ds(off, 16)]
                        r2 = rows2_v[b, tl, pl.ds(off, 16)]
                        v = (jnp.where(g1s > 0.0, r1 * g1s, zero)
                             + jnp.where(g2s > 0.0, r2 * g2s, zero))
                        outq_v[b, tl, pl.ds(off, 16)] = v
                    return _2

                return lax.fori_loop(0, D // 64, inner, _)

            lax.fori_loop(0, QT, body, 0)
            owrites[b] = pltpu.async_copy(
                outq_v.at[b], out_hbm.at[pl.ds(base + q * QT, QT)], semo[b])
            pend = nxt
        for w in owrites:
            if w is not None:
                w.wait()

    return dispatch, combine


def _ffn_body(xin_ref, w1_ref, b1_ref, w2_ref, b2_ref, out_ref):
    e = pl.program_id(0)
    xin = xin_ref[...]
    h = jnp.dot(xin, w1_ref[0], preferred_element_type=jnp.float32)
    h = jnp.maximum(h + b1_ref[pl.ds(e, 1), :], 0.0)
    out = jnp.dot(h, w2_ref[0], preferred_element_type=jnp.float32)
    out_ref[...] = out + b2_ref[pl.ds(e, 1), :]


def _ffn(exp_in, w1, b1, w2, b2):
    return pl.pallas_call(
        _ffn_body,
        grid=(E,),
        in_specs=[
            pl.BlockSpec((C, D), lambda e: (e, 0)),
            pl.BlockSpec((1, D, F), lambda e: (e, 0, 0)),
            pl.BlockSpec((E, F), lambda e: (0, 0)),
            pl.BlockSpec((1, F, D), lambda e: (e, 0, 0)),
            pl.BlockSpec((E, D), lambda e: (0, 0)),
        ],
        out_specs=pl.BlockSpec((C, D), lambda e: (e, 0)),
        out_shape=jax.ShapeDtypeStruct((NROWS, D), jnp.float32),
        compiler_params=pltpu.CompilerParams(
            vmem_limit_bytes=100 * 1024 * 1024),
    )(exp_in, w1, b1, w2, b2)


def kernel(hidden_states, Wg, W1, b1, W2, b2):
    x = hidden_states.reshape(S, D)
    s1, s2, g1, g2, loss = _routing(x, Wg.T)
    dispatch, combine = _sc_kernels()
    exp_in = dispatch(x, s1, s2)
    exp_out = _ffn(exp_in, W1, b1, W2, b2)
    out = combine(exp_out, s1, s2, g1, g2)
    return out.reshape(B, S, D), loss[0, 0]
